# Initial kernel scaffold; baseline (speedup 1.0000x reference)
import jax
import jax.numpy as jnp
from jax.experimental import pallas as pl


def kernel(x, edge_attr, edge_index, W1, b1, W2, b2, W3, b3, W4, b4):
    raise NotImplementedError("write your pallas kernel here")


# speedup vs baseline: 13.3250x; 13.3250x over previous
import functools

import jax
import jax.numpy as jnp
from jax import lax
from jax.experimental import pallas as pl
from jax.experimental.pallas import tpu as pltpu
from jax.experimental.pallas import tpu_sc as plsc

N = 10000
NPAD = 10240
E = 320000
NC, NS = 2, 16
NW = NC * NS
CHUNK = 128
NCHUNK = 80
EW = CHUNK * NCHUNK
EPAD = EW * NW
STRIPE = NPAD // NS
JUNK = NPAD - N

_MESH = plsc.VectorSubcoreMesh(
    core_axis_name="c", subcore_axis_name="s", num_cores=NC, num_subcores=NS)
_SC_PARAMS = pltpu.CompilerParams(use_tc_tiling_on_sc=False)


def _seg_body(src_hbm, dst_hbm, attr_hbm, z1_hbm, deg_out, xe_out,
              src_v, dst_v, attr_v, ones_v, deg_sh, xe_sh):
    c = lax.axis_index("c")
    s = lax.axis_index("s")
    wid = c * NS + s
    row = pl.ds(s * STRIPE, STRIPE)
    pltpu.sync_copy(z1_hbm.at[row], deg_sh.at[row])
    pltpu.sync_copy(z1_hbm.at[row], xe_sh.at[row])
    pltpu.sync_copy(src_hbm.at[wid], src_v)
    pltpu.sync_copy(dst_hbm.at[wid], dst_v)
    pltpu.sync_copy(attr_hbm.at[wid], attr_v)
    for i in range(CHUNK // 16):
        ones_v[pl.ds(i * 16, 16)] = jnp.full((16,), 1.0, jnp.float32)
    plsc.subcore_barrier()

    def step(j, carry):
        pltpu.sync_copy(ones_v, deg_sh.at[dst_v.at[j]], add=True)
        pltpu.sync_copy(attr_v.at[j], xe_sh.at[src_v.at[j]], add=True)
        return carry

    lax.fori_loop(0, NCHUNK, step, 0)
    plsc.subcore_barrier()
    pltpu.sync_copy(deg_sh.at[row], deg_out.at[c, row])
    pltpu.sync_copy(xe_sh.at[row], xe_out.at[c, row])


@functools.partial(
    pl.kernel,
    out_type=(jax.ShapeDtypeStruct((NC, NPAD), jnp.float32),
              jax.ShapeDtypeStruct((NC, NPAD), jnp.float32)),
    mesh=_MESH,
    scratch_types=[
        pltpu.VMEM((NCHUNK, CHUNK), jnp.int32),
        pltpu.VMEM((NCHUNK, CHUNK), jnp.int32),
        pltpu.VMEM((NCHUNK, CHUNK), jnp.float32),
        pltpu.VMEM((CHUNK,), jnp.float32),
        pltpu.VMEM_SHARED((NPAD,), jnp.float32),
        pltpu.VMEM_SHARED((NPAD,), jnp.float32),
    ],
    compiler_params=_SC_PARAMS,
)
def _seg(src_hbm, dst_hbm, attr_hbm, z1_hbm, deg_out, xe_out, *scratch):
    _seg_body(src_hbm, dst_hbm, attr_hbm, z1_hbm, deg_out, xe_out, *scratch)


def _conv_body(width, src_hbm, dst_hbm, hs_hbm, z_hbm, s_out,
               src_v, dst_v, rows_v, sem, acc_sh):
    c = lax.axis_index("c")
    s = lax.axis_index("s")
    wid = c * NS + s
    row = pl.ds(s * STRIPE, STRIPE)
    pltpu.sync_copy(z_hbm.at[row], acc_sh.at[row])
    pltpu.sync_copy(src_hbm.at[wid], src_v)
    pltpu.sync_copy(dst_hbm.at[wid], dst_v)
    plsc.subcore_barrier()

    def step(j, carry):
        pltpu.async_copy(hs_hbm.at[src_v.at[j]], rows_v, sem).wait()
        pltpu.sync_copy(rows_v, acc_sh.at[dst_v.at[j]], add=True)
        return carry

    lax.fori_loop(0, NCHUNK, step, 0)
    plsc.subcore_barrier()
    pltpu.sync_copy(acc_sh.at[row], s_out.at[c, row])


def _make_conv(width):
    if width == 1:
        out_t = jax.ShapeDtypeStruct((NC, NPAD), jnp.float32)
        rows_t = pltpu.VMEM((CHUNK,), jnp.float32)
        acc_t = pltpu.VMEM_SHARED((NPAD,), jnp.float32)
    else:
        out_t = jax.ShapeDtypeStruct((NC, NPAD, width), jnp.float32)
        rows_t = pltpu.VMEM((CHUNK, width), jnp.float32)
        acc_t = pltpu.VMEM_SHARED((NPAD, width), jnp.float32)

    @functools.partial(
        pl.kernel,
        out_type=out_t,
        mesh=_MESH,
        scratch_types=[
            pltpu.VMEM((NCHUNK, CHUNK), jnp.int32),
            pltpu.VMEM((NCHUNK, CHUNK), jnp.int32),
            rows_t,
            pltpu.SemaphoreType.DMA,
            acc_t,
        ],
        compiler_params=_SC_PARAMS,
    )
    def conv(src_hbm, dst_hbm, hs_hbm, z_hbm, s_out, *scratch):
        _conv_body(width, src_hbm, dst_hbm, hs_hbm, z_hbm, s_out, *scratch)

    return conv


_conv32 = _make_conv(32)
_conv1 = _make_conv(1)


_RB = 512
_GRID = NPAD // _RB


def _tc_a_body(x_ref, deg2_ref, xe2_ref, w1a_ref, w1b_ref,
               dinv_ref, xe_ref, hs1_ref):
    deg = deg2_ref[0] + deg2_ref[1] + 1.0
    dinv = lax.rsqrt(deg)
    xe = xe2_ref[0] + xe2_ref[1]
    h = jnp.dot(x_ref[...], w1a_ref[...],
                preferred_element_type=jnp.float32) + xe * w1b_ref[...]
    dinv_ref[...] = dinv
    xe_ref[...] = xe
    hs1_ref[...] = dinv * h


_tc_a = pl.pallas_call(
    _tc_a_body,
    grid=(_GRID,),
    in_specs=[
        pl.BlockSpec((_RB, 128), lambda i: (i, 0)),
        pl.BlockSpec((NC, _RB, 1), lambda i: (0, i, 0)),
        pl.BlockSpec((NC, _RB, 1), lambda i: (0, i, 0)),
        pl.BlockSpec((128, 32), lambda i: (0, 0)),
        pl.BlockSpec((1, 32), lambda i: (0, 0)),
    ],
    out_specs=[
        pl.BlockSpec((_RB, 1), lambda i: (i, 0)),
        pl.BlockSpec((_RB, 1), lambda i: (i, 0)),
        pl.BlockSpec((_RB, 32), lambda i: (i, 0)),
    ],
    out_shape=[
        jax.ShapeDtypeStruct((NPAD, 1), jnp.float32),
        jax.ShapeDtypeStruct((NPAD, 1), jnp.float32),
        jax.ShapeDtypeStruct((NPAD, 32), jnp.float32),
    ],
)


def _tc_b_body(s2_ref, hs_ref, dinv_ref, w_ref, b_ref, out_ref, hsn_ref):
    dinv = dinv_ref[...]
    out = dinv * (s2_ref[0] + s2_ref[1] + hs_ref[...]) + b_ref[...]
    out_ref[...] = out
    hsn_ref[...] = dinv * jnp.dot(out, w_ref[...],
                                  preferred_element_type=jnp.float32)


def _make_tc_b(w_out):
    return pl.pallas_call(
        _tc_b_body,
        grid=(_GRID,),
        in_specs=[
            pl.BlockSpec((NC, _RB, 32), lambda i: (0, i, 0)),
            pl.BlockSpec((_RB, 32), lambda i: (i, 0)),
            pl.BlockSpec((_RB, 1), lambda i: (i, 0)),
            pl.BlockSpec((32, w_out), lambda i: (0, 0)),
            pl.BlockSpec((1, 32), lambda i: (0, 0)),
        ],
        out_specs=[
            pl.BlockSpec((_RB, 32), lambda i: (i, 0)),
            pl.BlockSpec((_RB, w_out), lambda i: (i, 0)),
        ],
        out_shape=[
            jax.ShapeDtypeStruct((NPAD, 32), jnp.float32),
            jax.ShapeDtypeStruct((NPAD, w_out), jnp.float32),
        ],
    )


_tc_b32 = _make_tc_b(32)
_tc_b1 = _make_tc_b(1)


def _tc_final_body(s2_ref, hs_ref, dinv_ref, b_ref, out_ref):
    out_ref[...] = dinv_ref[...] * (s2_ref[0] + s2_ref[1] + hs_ref[...]) + b_ref[...]


_tc_final = pl.pallas_call(
    _tc_final_body,
    grid=(_GRID,),
    in_specs=[
        pl.BlockSpec((NC, _RB, 1), lambda i: (0, i, 0)),
        pl.BlockSpec((_RB, 1), lambda i: (i, 0)),
        pl.BlockSpec((_RB, 1), lambda i: (i, 0)),
        pl.BlockSpec((1, 1), lambda i: (0, 0)),
    ],
    out_specs=pl.BlockSpec((_RB, 1), lambda i: (i, 0)),
    out_shape=jax.ShapeDtypeStruct((NPAD, 1), jnp.float32),
)


def kernel(x, edge_attr, edge_index, W1, b1, W2, b2, W3, b3, W4, b4):
    src, dst = edge_index[0], edge_index[1]
    pad = EPAD - E
    srcp = jnp.concatenate([src, jnp.zeros((pad,), jnp.int32)])
    dstp = jnp.concatenate(
        [dst, N + (jnp.arange(pad, dtype=jnp.int32) % JUNK)])
    attrp = jnp.concatenate([edge_attr[:, 0], jnp.zeros((pad,), jnp.float32)])
    src_r = srcp.reshape(NW, NCHUNK, CHUNK)
    dst_r = dstp.reshape(NW, NCHUNK, CHUNK)
    attr_r = attrp.reshape(NW, NCHUNK, CHUNK)
    z1 = jnp.zeros((NPAD,), jnp.float32)
    z32 = jnp.zeros((NPAD, 32), jnp.float32)
    x_p = jnp.pad(x, ((0, NPAD - N), (0, 0)))

    deg2, xe2 = _seg(src_r, dst_r, attr_r, z1)
    dinv, xe, hs1 = _tc_a(x_p, deg2[:, :, None], xe2[:, :, None],
                          W1[:128], W1[128:129])

    s1 = _conv32(src_r, dst_r, hs1, z32)
    out1, hs2 = _tc_b32(s1, hs1, dinv, W2, b1.reshape(1, 32))
    s2 = _conv32(src_r, dst_r, hs2, z32)
    out2, hs3 = _tc_b32(s2, hs2, dinv, W3, b2.reshape(1, 32))
    s3 = _conv32(src_r, dst_r, hs3, z32)
    out3, hs4 = _tc_b1(s3, hs3, dinv, W4, b3.reshape(1, 32))
    s4 = _conv1(src_r, dst_r, hs4[:, 0], z1)
    out4 = _tc_final(s4[:, :, None], hs4, dinv, b4.reshape(1, 1))

    return jnp.concatenate(
        [x, xe[:N], out1[:N], out2[:N], out3[:N], out4[:N]], axis=1)
